# Initial kernel scaffold; baseline (speedup 1.0000x reference)
import jax
import jax.numpy as jnp
from jax.experimental import pallas as pl


def kernel(idx, targets, emb):
    raise NotImplementedError("write your pallas kernel here")


# speedup vs baseline: 1.3929x; 1.3929x over previous
import functools

import jax
import jax.numpy as jnp
from jax import lax
from jax.experimental import pallas as pl
from jax.experimental.pallas import tpu as pltpu
from jax.experimental.pallas import tpu_sc as plsc

V = 1000
N = 51200
NC, NS, L = 2, 16, 16
NW = NC * NS
ROWS_PER_W = N // NW
BC = 64
CHUNKS = ROWS_PER_W // BC


def _prep_body(emb_ref, idx_ref, t_ref, lse_ref, pidx_ref):
    x = emb_ref[...]
    m = jnp.max(x, axis=1)
    s = jnp.sum(jnp.exp(x - m[:, None]), axis=1)
    lse_ref[...] = m + jnp.log(s)
    pidx_ref[...] = idx_ref[...] * V + t_ref[...]


def _tc_prep(emb, idx_flat, t_flat):
    return pl.pallas_call(
        _prep_body,
        out_shape=[
            jax.ShapeDtypeStruct((V,), jnp.float32),
            jax.ShapeDtypeStruct((N,), jnp.int32),
        ],
    )(emb, idx_flat, t_flat)


@functools.partial(
    pl.kernel,
    mesh=plsc.VectorSubcoreMesh(core_axis_name="c", subcore_axis_name="s"),
    out_type=[
        jax.ShapeDtypeStruct((N, V), jnp.float32),
        jax.ShapeDtypeStruct((NW, L), jnp.float32),
    ],
    scratch_types=[
        pltpu.VMEM((ROWS_PER_W,), jnp.int32),
        pltpu.VMEM((ROWS_PER_W,), jnp.int32),
        pltpu.VMEM((BC, V), jnp.float32),
        pltpu.VMEM((BC,), jnp.float32),
        pltpu.VMEM((BC,), jnp.float32),
        pltpu.VMEM((L,), jnp.float32),
        pltpu.SemaphoreType.DMA,
        pltpu.SemaphoreType.DMA,
        pltpu.SemaphoreType.DMA,
    ],
    compiler_params=pltpu.CompilerParams(use_tc_tiling_on_sc=False),
)
def _sc_gather(idx_hbm, pidx_hbm, emb_hbm, embf_hbm, lse_hbm,
               out_hbm, part_hbm,
               idx_v, pidx_v, rows_v, picked_v, lsev_v, acc_v,
               sem, sem2, sem3):
    wid = lax.axis_index("s") * NC + lax.axis_index("c")
    base = wid * ROWS_PER_W
    pltpu.sync_copy(idx_hbm.at[pl.ds(base, ROWS_PER_W)], idx_v)
    pltpu.sync_copy(pidx_hbm.at[pl.ds(base, ROWS_PER_W)], pidx_v)

    def body(ci, acc):
        r0 = pl.multiple_of(ci * BC, BC)
        rows_cp = pltpu.async_copy(
            emb_hbm.at[idx_v.at[pl.ds(r0, BC)]], rows_v, sem)
        pk_cp = pltpu.async_copy(
            embf_hbm.at[pidx_v.at[pl.ds(r0, BC)]], picked_v, sem2)
        ls_cp = pltpu.async_copy(
            lse_hbm.at[idx_v.at[pl.ds(r0, BC)]], lsev_v, sem3)
        rows_cp.wait()
        pltpu.sync_copy(rows_v, out_hbm.at[pl.ds(base + r0, BC)])
        pk_cp.wait()
        ls_cp.wait()
        for g in range(BC // L):
            acc = acc + (lsev_v[pl.ds(g * L, L)] - picked_v[pl.ds(g * L, L)])
        return acc

    acc = lax.fori_loop(0, CHUNKS, body, jnp.zeros((L,), jnp.float32))
    acc_v[...] = acc
    pltpu.sync_copy(acc_v, part_hbm.at[wid])


def kernel(idx, targets, emb):
    idx_flat = idx.reshape(-1)
    t_flat = targets.reshape(-1)
    lse, pidx = _tc_prep(emb, idx_flat, t_flat)
    embf = jnp.pad(emb.reshape(-1), (0, 8))
    logits, part = _sc_gather(idx_flat, pidx, emb, embf, lse)
    loss = jnp.sum(part) / float(N)
    return (logits, loss)
